# speedup vs baseline: 1.8453x; 1.8453x over previous
import functools

import jax
import jax.numpy as jnp
from jax import lax
from jax.experimental import pallas as pl
from jax.experimental.pallas import tpu as pltpu
from jax.experimental.pallas import tpu_sc as plsc

EPS = 1e-12
_NW = 32
_CHUNK = 32


def _sc_gather(table, idx):
    tok = idx.shape[0]
    h = table.shape[1]
    bpw = tok // _NW

    mesh = plsc.VectorSubcoreMesh(core_axis_name="c", subcore_axis_name="s")

    @functools.partial(
        pl.kernel,
        mesh=mesh,
        out_type=jax.ShapeDtypeStruct((tok, h), jnp.float32),
        scratch_types=[
            pltpu.VMEM((_CHUNK,), jnp.int32),
            pltpu.VMEM((_CHUNK, h), jnp.float32),
            pltpu.SemaphoreType.DMA,
        ],
    )
    def gather_kernel(table_hbm, idx_hbm, out_hbm, idx_v, rows_v, sem):
        wid = lax.axis_index("s") * 2 + lax.axis_index("c")
        base = wid * bpw

        @pl.loop(0, bpw, step=_CHUNK)
        def _(j):
            b = base + j
            pltpu.sync_copy(idx_hbm.at[pl.ds(b, _CHUNK)], idx_v)
            pltpu.async_copy(table_hbm.at[idx_v], rows_v, sem).wait()
            pltpu.sync_copy(rows_v, out_hbm.at[pl.ds(b, _CHUNK)])

    return gather_kernel(table, idx)


def _ln_body(g_ref, pos_ref, t_ref, vecs_ref, out_ref):
    x = (g_ref[...] + pos_ref[...]
         + t_ref[...] * vecs_ref[1:2, :] + vecs_ref[0:1, :])
    u = jnp.mean(x, axis=-1, keepdims=True)
    xc = x - u
    s = jnp.mean(xc * xc, axis=-1, keepdims=True)
    y = xc * lax.rsqrt(s + EPS)
    out_ref[...] = y * vecs_ref[2:3, :] + vecs_ref[3:4, :]


def _fused_ln(gathered, pos, t, vecs, block_rows=256):
    tok, h = gathered.shape
    s_rows = pos.shape[0]
    blocks_per_seq = s_rows // block_rows
    return pl.pallas_call(
        _ln_body,
        out_shape=jax.ShapeDtypeStruct((tok, h), jnp.float32),
        grid=(tok // block_rows,),
        in_specs=[
            pl.BlockSpec((block_rows, h), lambda b: (b, 0)),
            pl.BlockSpec((block_rows, h), lambda b: (b % blocks_per_seq, 0)),
            pl.BlockSpec((block_rows, 1), lambda b: (b, 0)),
            pl.BlockSpec((4, h), lambda b: (0, 0)),
        ],
        out_specs=pl.BlockSpec((block_rows, h), lambda b: (b, 0)),
    )(gathered, pos, t, vecs)


def kernel(input_ids, token_type_ids, word_emb, pos_emb, type_emb,
           ln_gamma, ln_beta):
    b, s = input_ids.shape
    h = word_emb.shape[1]
    tok = b * s

    ids = input_ids.reshape(tok).astype(jnp.int32)
    gathered = _sc_gather(word_emb, ids)

    t = token_type_ids.reshape(tok, 1).astype(jnp.float32)
    vecs = jnp.stack(
        [type_emb[0], type_emb[1] - type_emb[0], ln_gamma, ln_beta])
    out = _fused_ln(gathered, pos_emb[:s], t, vecs)
    return out.reshape(b, s, h)


# speedup vs baseline: 2.2976x; 1.2451x over previous
import functools

import jax
import jax.numpy as jnp
from jax import lax
from jax.experimental import pallas as pl
from jax.experimental.pallas import tpu as pltpu
from jax.experimental.pallas import tpu_sc as plsc

EPS = 1e-12
_NW = 32
_CHUNK = 32
_NBUF = 4


def _sc_gather(table, idx):
    tok = idx.shape[0]
    h = table.shape[1]
    bpw = tok // _NW
    nchunks = bpw // _CHUNK

    mesh = plsc.VectorSubcoreMesh(core_axis_name="c", subcore_axis_name="s")

    @functools.partial(
        pl.kernel,
        mesh=mesh,
        out_type=jax.ShapeDtypeStruct((tok, h), jnp.float32),
        scratch_types=(
            [pltpu.VMEM((bpw,), jnp.int32)]
            + [pltpu.VMEM((_CHUNK, h), jnp.float32) for _ in range(_NBUF)]
            + [pltpu.SemaphoreType.DMA for _ in range(2 * _NBUF)]
        ),
    )
    def gather_kernel(table_hbm, idx_hbm, out_hbm, idx_v, *rest):
        bufs = rest[:_NBUF]
        gsems = rest[_NBUF:2 * _NBUF]
        wsems = rest[2 * _NBUF:]
        wid = lax.axis_index("s") * 2 + lax.axis_index("c")
        base = wid * bpw
        pltpu.sync_copy(idx_hbm.at[pl.ds(base, bpw)], idx_v)

        def gather_start(g):
            b = g % _NBUF
            return pltpu.async_copy(
                table_hbm.at[idx_v.at[pl.ds(g * _CHUNK, _CHUNK)]],
                bufs[b], gsems[b])

        gcp = {g: gather_start(g) for g in range(min(_NBUF, nchunks))}
        wcp = {}
        for g in range(nchunks):
            b = g % _NBUF
            gcp[g].wait()
            wcp[g] = pltpu.async_copy(
                bufs[b], out_hbm.at[pl.ds(base + g * _CHUNK, _CHUNK)],
                wsems[b])
            nxt = g + _NBUF
            if nxt < nchunks:
                wcp[g].wait()
                gcp[nxt] = gather_start(nxt)
        for g in range(max(0, nchunks - _NBUF), nchunks):
            wcp[g].wait()

    return gather_kernel(table, idx)


def _ln_body(g_ref, pos_ref, t_ref, vecs_ref, out_ref):
    x = (g_ref[...] + pos_ref[...]
         + t_ref[...] * vecs_ref[1:2, :] + vecs_ref[0:1, :])
    u = jnp.mean(x, axis=-1, keepdims=True)
    xc = x - u
    s = jnp.mean(xc * xc, axis=-1, keepdims=True)
    y = xc * lax.rsqrt(s + EPS)
    out_ref[...] = y * vecs_ref[2:3, :] + vecs_ref[3:4, :]


def _fused_ln(gathered, pos, t, vecs, block_rows=512):
    tok, h = gathered.shape
    s_rows = pos.shape[0]
    blocks_per_seq = s_rows // block_rows
    return pl.pallas_call(
        _ln_body,
        out_shape=jax.ShapeDtypeStruct((tok, h), jnp.float32),
        grid=(tok // block_rows,),
        in_specs=[
            pl.BlockSpec((block_rows, h), lambda b: (b, 0)),
            pl.BlockSpec((block_rows, h), lambda b: (b % blocks_per_seq, 0)),
            pl.BlockSpec((block_rows, 1), lambda b: (b, 0)),
            pl.BlockSpec((4, h), lambda b: (0, 0)),
        ],
        out_specs=pl.BlockSpec((block_rows, h), lambda b: (b, 0)),
    )(gathered, pos, t, vecs)


def kernel(input_ids, token_type_ids, word_emb, pos_emb, type_emb,
           ln_gamma, ln_beta):
    b, s = input_ids.shape
    h = word_emb.shape[1]
    tok = b * s

    ids = input_ids.reshape(tok).astype(jnp.int32)
    gathered = _sc_gather(word_emb, ids)

    t = token_type_ids.reshape(tok, 1).astype(jnp.float32)
    vecs = jnp.stack(
        [type_emb[0], type_emb[1] - type_emb[0], ln_gamma, ln_beta])
    out = _fused_ln(gathered, pos_emb[:s], t, vecs)
    return out.reshape(b, s, h)


# speedup vs baseline: 2.3799x; 1.0358x over previous
import functools

import jax
import jax.numpy as jnp
from jax import lax
from jax.experimental import pallas as pl
from jax.experimental.pallas import tpu as pltpu
from jax.experimental.pallas import tpu_sc as plsc

EPS = 1e-12
_NW = 32
_CHUNK = 32
_NBUF = 4


def _sc_gather(table, idx):
    tok = idx.shape[0]
    h = table.shape[1]
    bpw = tok // _NW
    nchunks = bpw // _CHUNK

    mesh = plsc.VectorSubcoreMesh(core_axis_name="c", subcore_axis_name="s")

    @functools.partial(
        pl.kernel,
        mesh=mesh,
        out_type=jax.ShapeDtypeStruct((tok, h), jnp.float32),
        scratch_types=(
            [pltpu.VMEM((bpw,), jnp.int32)]
            + [pltpu.VMEM((_CHUNK, h), jnp.float32) for _ in range(_NBUF)]
            + [pltpu.SemaphoreType.DMA for _ in range(2 * _NBUF)]
        ),
    )
    def gather_kernel(table_hbm, idx_hbm, out_hbm, idx_v, *rest):
        bufs = rest[:_NBUF]
        gsems = rest[_NBUF:2 * _NBUF]
        wsems = rest[2 * _NBUF:]
        wid = lax.axis_index("s") * 2 + lax.axis_index("c")
        base = wid * bpw
        pltpu.sync_copy(idx_hbm.at[pl.ds(base, bpw)], idx_v)

        def gather_start(g):
            b = g % _NBUF
            return pltpu.async_copy(
                table_hbm.at[idx_v.at[pl.ds(g * _CHUNK, _CHUNK)]],
                bufs[b], gsems[b])

        gcp = {g: gather_start(g) for g in range(min(_NBUF, nchunks))}
        wcp = {}
        for g in range(nchunks):
            b = g % _NBUF
            gcp[g].wait()
            wcp[g] = pltpu.async_copy(
                bufs[b], out_hbm.at[pl.ds(base + g * _CHUNK, _CHUNK)],
                wsems[b])
            nxt = g + _NBUF
            if nxt < nchunks:
                wcp[g].wait()
                gcp[nxt] = gather_start(nxt)
        for g in range(max(0, nchunks - _NBUF), nchunks):
            wcp[g].wait()

    return gather_kernel(table, idx)


def _ln_body(g_ref, pos_ref, t_ref, type_ref, gamma_ref, beta_ref, out_ref):
    delta = type_ref[1:2, :] - type_ref[0:1, :]
    x = g_ref[...] + pos_ref[...] + t_ref[...] * delta + type_ref[0:1, :]
    u = jnp.mean(x, axis=-1, keepdims=True)
    xc = x - u
    s = jnp.mean(xc * xc, axis=-1, keepdims=True)
    y = xc * lax.rsqrt(s + EPS)
    out_ref[...] = y * gamma_ref[...] + beta_ref[...]


def _fused_ln(gathered, pos, t, type_emb, gamma, beta, block_rows=512):
    tok, h = gathered.shape
    s_rows = pos.shape[0]
    bps = s_rows // block_rows
    nb = tok // s_rows
    row_block = lambda j, i: (i * bps + j, 0)
    return pl.pallas_call(
        _ln_body,
        out_shape=jax.ShapeDtypeStruct((tok, h), jnp.float32),
        grid=(bps, nb),
        in_specs=[
            pl.BlockSpec((block_rows, h), row_block),
            pl.BlockSpec((block_rows, h), lambda j, i: (j, 0)),
            pl.BlockSpec((block_rows, 1), row_block),
            pl.BlockSpec((2, h), lambda j, i: (0, 0)),
            pl.BlockSpec((1, h), lambda j, i: (0, 0)),
            pl.BlockSpec((1, h), lambda j, i: (0, 0)),
        ],
        out_specs=pl.BlockSpec((block_rows, h), row_block),
    )(gathered, pos, t, type_emb, gamma.reshape(1, h), beta.reshape(1, h))


def kernel(input_ids, token_type_ids, word_emb, pos_emb, type_emb,
           ln_gamma, ln_beta):
    b, s = input_ids.shape
    h = word_emb.shape[1]
    tok = b * s

    ids = input_ids.reshape(tok).astype(jnp.int32)
    gathered = _sc_gather(word_emb, ids)

    t = token_type_ids.reshape(tok, 1).astype(jnp.float32)
    out = _fused_ln(gathered, pos_emb[:s], t, type_emb, ln_gamma, ln_beta)
    return out.reshape(b, s, h)


# speedup vs baseline: 2.3975x; 1.0074x over previous
import functools

import jax
import jax.numpy as jnp
from jax import lax
from jax.experimental import pallas as pl
from jax.experimental.pallas import tpu as pltpu
from jax.experimental.pallas import tpu_sc as plsc

EPS = 1e-12
_NW = 32
_CHUNK = 32
_NBUF = 4


def _sc_gather(table, idx):
    bdim, sdim = idx.shape
    tok = bdim * sdim
    h = table.shape[1]
    bpw = tok // _NW
    nchunks = bpw // _CHUNK
    wpr = sdim // bpw

    mesh = plsc.VectorSubcoreMesh(core_axis_name="c", subcore_axis_name="s")

    @functools.partial(
        pl.kernel,
        mesh=mesh,
        out_type=jax.ShapeDtypeStruct((tok, h), jnp.float32),
        scratch_types=(
            [pltpu.VMEM((bpw,), jnp.int32)]
            + [pltpu.VMEM((_CHUNK, h), jnp.float32) for _ in range(_NBUF)]
            + [pltpu.SemaphoreType.DMA for _ in range(2 * _NBUF)]
        ),
    )
    def gather_kernel(table_hbm, idx_hbm, out_hbm, idx_v, *rest):
        bufs = rest[:_NBUF]
        gsems = rest[_NBUF:2 * _NBUF]
        wsems = rest[2 * _NBUF:]
        wid = lax.axis_index("s") * 2 + lax.axis_index("c")
        base = wid * bpw
        pltpu.sync_copy(
            idx_hbm.at[wid // wpr, pl.ds((wid % wpr) * bpw, bpw)], idx_v)

        def gather_start(g):
            b = g % _NBUF
            return pltpu.async_copy(
                table_hbm.at[idx_v.at[pl.ds(g * _CHUNK, _CHUNK)]],
                bufs[b], gsems[b])

        gcp = {g: gather_start(g) for g in range(min(_NBUF, nchunks))}
        wcp = {}
        for g in range(nchunks):
            b = g % _NBUF
            gcp[g].wait()
            wcp[g] = pltpu.async_copy(
                bufs[b], out_hbm.at[pl.ds(base + g * _CHUNK, _CHUNK)],
                wsems[b])
            nxt = g + _NBUF
            if nxt < nchunks:
                wcp[g].wait()
                gcp[nxt] = gather_start(nxt)
        for g in range(max(0, nchunks - _NBUF), nchunks):
            wcp[g].wait()

    return gather_kernel(table, idx)


def _ln_body(g_ref, pos_ref, t_ref, type_ref, gamma_ref, beta_ref, out_ref):
    h = g_ref.shape[-1]
    delta = type_ref[1:2, :] - type_ref[0:1, :]
    x = g_ref[...] + pos_ref[...] + t_ref[...] * delta + type_ref[0:1, :]
    u = jnp.mean(x, axis=-1, keepdims=True)
    xc = x - u
    s = jnp.mean(xc * xc, axis=-1, keepdims=True)
    y = xc * lax.rsqrt(s + EPS)
    out_ref[...] = y * gamma_ref[...] + beta_ref[...]


def _fused_ln(gathered, pos, t, type_emb, gamma, beta, block_rows=512):
    tok, h = gathered.shape
    s_rows = pos.shape[0]
    bps = s_rows // block_rows
    nb = tok // s_rows
    row_block = lambda j, i: (i * bps + j, 0)
    return pl.pallas_call(
        _ln_body,
        out_shape=jax.ShapeDtypeStruct((tok, h), jnp.float32),
        grid=(bps, nb),
        in_specs=[
            pl.BlockSpec((block_rows, h), row_block),
            pl.BlockSpec((block_rows, h), lambda j, i: (j, 0)),
            pl.BlockSpec((block_rows, 1), row_block),
            pl.BlockSpec((2, h), lambda j, i: (0, 0)),
            pl.BlockSpec((1, h), lambda j, i: (0, 0)),
            pl.BlockSpec((1, h), lambda j, i: (0, 0)),
        ],
        out_specs=pl.BlockSpec((block_rows, h), row_block),
    )(gathered, pos, t, type_emb, gamma.reshape(1, h), beta.reshape(1, h))


def kernel(input_ids, token_type_ids, word_emb, pos_emb, type_emb,
           ln_gamma, ln_beta):
    b, s = input_ids.shape
    h = word_emb.shape[1]
    tok = b * s

    gathered = _sc_gather(word_emb, input_ids.astype(jnp.int32))

    t = token_type_ids.reshape(tok, 1).astype(jnp.float32)
    out = _fused_ln(gathered, pos_emb[:s], t, type_emb, ln_gamma, ln_beta)
    return out.reshape(b, s, h)


# speedup vs baseline: 2.5527x; 1.0648x over previous
import functools

import jax
import jax.numpy as jnp
from jax import lax
from jax.experimental import pallas as pl
from jax.experimental.pallas import tpu as pltpu
from jax.experimental.pallas import tpu_sc as plsc

EPS = 1e-12
_NW = 32
_CHUNK = 32
_NBUF = 4


def _sc_gather(table, idx):
    bdim, sdim = idx.shape
    tok = bdim * sdim
    h = table.shape[1]
    bpw = tok // _NW
    nchunks = bpw // _CHUNK
    wpr = sdim // bpw

    mesh = plsc.VectorSubcoreMesh(core_axis_name="c", subcore_axis_name="s")

    @functools.partial(
        pl.kernel,
        mesh=mesh,
        out_type=jax.ShapeDtypeStruct((tok, h), jnp.float32),
        scratch_types=(
            [pltpu.VMEM((bpw,), jnp.int32)]
            + [pltpu.VMEM((_CHUNK, h), jnp.float32) for _ in range(_NBUF)]
            + [pltpu.SemaphoreType.DMA for _ in range(2 * _NBUF)]
        ),
    )
    def gather_kernel(table_hbm, idx_hbm, out_hbm, idx_v, *rest):
        bufs = rest[:_NBUF]
        gsems = rest[_NBUF:2 * _NBUF]
        wsems = rest[2 * _NBUF:]
        wid = lax.axis_index("s") * 2 + lax.axis_index("c")
        base = wid * bpw
        pltpu.sync_copy(
            idx_hbm.at[wid // wpr, pl.ds((wid % wpr) * bpw, bpw)], idx_v)

        def gather_start(g):
            b = g % _NBUF
            return pltpu.async_copy(
                table_hbm.at[idx_v.at[pl.ds(g * _CHUNK, _CHUNK)]],
                bufs[b], gsems[b])

        gcp = {g: gather_start(g) for g in range(min(_NBUF, nchunks))}
        wcp = {}
        for g in range(nchunks):
            b = g % _NBUF
            gcp[g].wait()
            wcp[g] = pltpu.async_copy(
                bufs[b], out_hbm.at[pl.ds(base + g * _CHUNK, _CHUNK)],
                wsems[b])
            nxt = g + _NBUF
            if nxt < nchunks:
                wcp[g].wait()
                gcp[nxt] = gather_start(nxt)
        for g in range(max(0, nchunks - _NBUF), nchunks):
            wcp[g].wait()

    return gather_kernel(table, idx)


def _ln_body(g_ref, pos_ref, t_ref, type_ref, gamma_ref, beta_ref, out_ref):
    h = g_ref.shape[-1]
    delta = type_ref[1:2, :] - type_ref[0:1, :]
    x = g_ref[...] + pos_ref[...] + t_ref[...] * delta + type_ref[0:1, :]
    u = jnp.mean(x, axis=-1, keepdims=True)
    xc = x - u
    s = jnp.mean(xc * xc, axis=-1, keepdims=True)
    y = xc * lax.rsqrt(s + EPS)
    out_ref[...] = y * gamma_ref[...] + beta_ref[...]


def _fused_ln(gathered, pos, t, type_emb, gamma, beta, block_rows=1024):
    tok, h = gathered.shape
    s_rows = pos.shape[0]
    bps = s_rows // block_rows
    nb = tok // s_rows
    row_block = lambda j, i: (i * bps + j, 0)
    return pl.pallas_call(
        _ln_body,
        out_shape=jax.ShapeDtypeStruct((tok, h), jnp.float32),
        grid=(bps, nb),
        in_specs=[
            pl.BlockSpec((block_rows, h), row_block),
            pl.BlockSpec((block_rows, h), lambda j, i: (j, 0)),
            pl.BlockSpec((block_rows, 1), row_block),
            pl.BlockSpec((2, h), lambda j, i: (0, 0)),
            pl.BlockSpec((1, h), lambda j, i: (0, 0)),
            pl.BlockSpec((1, h), lambda j, i: (0, 0)),
        ],
        out_specs=pl.BlockSpec((block_rows, h), row_block),
    )(gathered, pos, t, type_emb, gamma.reshape(1, h), beta.reshape(1, h))


def kernel(input_ids, token_type_ids, word_emb, pos_emb, type_emb,
           ln_gamma, ln_beta):
    b, s = input_ids.shape
    h = word_emb.shape[1]
    tok = b * s

    gathered = _sc_gather(word_emb, input_ids.astype(jnp.int32))

    t = token_type_ids.reshape(tok, 1).astype(jnp.float32)
    out = _fused_ln(gathered, pos_emb[:s], t, type_emb, ln_gamma, ln_beta)
    return out.reshape(b, s, h)
